# baseline (speedup 1.0000x reference)
import functools

import jax
import jax.numpy as jnp
from jax import lax
from jax.experimental import pallas as pl
from jax.experimental.pallas import tpu as pltpu
from jax.experimental.pallas import tpu_sc as plsc

_N = 10000
_E = 160000
_IN = 32
_OUT = 32
_NEF = 16
_HID = 64
_EPS = 1e-5

_NW = 32
_CHUNK = 128
_NCH = 40
_EPW = _CHUNK * _NCH
_EP = _EPW * _NW
_NBUF = 7
_SBUF = 2
_NP = 10240
_ROWS_PER_SUB = _NP // 16
_TCB = 512


def _gather_body(node4_hbm, dst4_hbm, x_hbm, idx_v, rows_v, sem_in, sem_out):
    c = lax.axis_index("c")
    s = lax.axis_index("s")
    wid = s * 2 + c
    pltpu.sync_copy(dst4_hbm.at[wid], idx_v)
    base = wid * _EPW
    cin = [None] * _NCH
    cout = [None] * _NCH
    for j in range(_NBUF):
        cin[j] = pltpu.async_copy(node4_hbm.at[idx_v.at[j]], rows_v.at[j % _NBUF],
                                  sem_in)
    for j in range(_NCH):
        cin[j].wait()
        cout[j] = pltpu.async_copy(
            rows_v.at[j % _NBUF], x_hbm.at[pl.ds(base + j * _CHUNK, _CHUNK)], sem_out)
        nj = j + _NBUF
        if nj < _NCH:
            cout[j].wait()
            cin[nj] = pltpu.async_copy(node4_hbm.at[idx_v.at[nj]],
                                       rows_v.at[nj % _NBUF], sem_in)
    for j in range(_NCH - _NBUF, _NCH):
        cout[j].wait()


def _scatter_body(tpc_hbm, src_hbm, zinit_hbm, out_hbm, idx_v, rows_v, acc_sh, sem):
    c = lax.axis_index("c")
    s = lax.axis_index("s")
    wid = s * 2 + c
    pltpu.sync_copy(zinit_hbm.at[pl.ds(s * _ROWS_PER_SUB, _ROWS_PER_SUB)],
                    acc_sh.at[pl.ds(s * _ROWS_PER_SUB, _ROWS_PER_SUB)])
    pltpu.sync_copy(src_hbm.at[wid], idx_v)
    plsc.subcore_barrier()
    base = wid * _EPW
    for g in range(_NCH // _SBUF):
        cps = []
        for b in range(_SBUF):
            j = g * _SBUF + b
            cps.append(pltpu.async_copy(
                tpc_hbm.at[pl.ds(base + j * _CHUNK, _CHUNK)], rows_v.at[b], sem))
        for b in range(_SBUF):
            j = g * _SBUF + b
            cps[b].wait()
            pltpu.sync_copy(rows_v.at[b], acc_sh.at[idx_v.at[j]], add=True)
    plsc.subcore_barrier()
    pltpu.sync_copy(acc_sh.at[pl.ds(s * _ROWS_PER_SUB, _ROWS_PER_SUB)],
                    out_hbm.at[c, pl.ds(s * _ROWS_PER_SUB, _ROWS_PER_SUB)])


def _tc_body(ea_ref, sh_ref, off_ref, xq_ref, w1_ref, b1_ref, w2_ref, b2_ref,
             rep_ref, out_ref):
    xq = xq_ref[...]
    off = off_ref[...]
    x = ((off == 0) * xq[:, 0:32] + (off == 1) * xq[:, 32:64]
         + (off == 2) * xq[:, 64:96] + (off == 3) * xq[:, 96:128])
    h = jnp.maximum(
        jnp.dot(ea_ref[...], w1_ref[...], preferred_element_type=jnp.float32)
        + b1_ref[...], 0.0)
    wb = jnp.dot(h, w2_ref[...], preferred_element_type=jnp.float32) + b2_ref[...]
    xf = jnp.dot(x, rep_ref[...], preferred_element_type=jnp.float32)
    p = xf * wb
    p = p[:, :512] + p[:, 512:]
    p = p[:, :256] + p[:, 256:]
    p = p[:, :128] + p[:, 128:]
    p = p[:, :64] + p[:, 64:]
    p = p[:, :32] + p[:, 32:]
    tp = p * sh_ref[...]
    i = pl.program_id(0)
    valid = ((lax.broadcasted_iota(jnp.int32, (_TCB, 16), 0) + i * _TCB) < _E
             ).astype(jnp.float32)
    out_ref[...] = jnp.concatenate([tp, valid, jnp.zeros((_TCB, 80), jnp.float32)], axis=1)


def _fin_body(parts_ref, node_ref, gamma_ref, beta_ref, out_ref):
    sums = parts_ref[0, :_N, :_OUT] + parts_ref[1, :_N, :_OUT]
    cnt = parts_ref[0, :_N, _OUT:_OUT + 1] + parts_ref[1, :_N, _OUT:_OUT + 1]
    o = sums / jnp.maximum(cnt, 1.0) + node_ref[...]
    mu = jnp.mean(o, axis=0, keepdims=True)
    var = jnp.mean((o - mu) ** 2, axis=0, keepdims=True)
    out_ref[...] = (o - mu) * lax.rsqrt(var + _EPS) * gamma_ref[...] + beta_ref[...]


_sc_mesh = plsc.VectorSubcoreMesh(core_axis_name="c", subcore_axis_name="s")

_gather_call = functools.partial(
    pl.kernel,
    out_type=jax.ShapeDtypeStruct((_EP, 128), jnp.float32),
    mesh=_sc_mesh,
    scratch_types=[
        pltpu.VMEM((_NCH, _CHUNK), jnp.int32),
        pltpu.VMEM((_NBUF, _CHUNK, 128), jnp.float32),
        pltpu.SemaphoreType.DMA,
        pltpu.SemaphoreType.DMA,
    ],
)(_gather_body)

_scatter_call = functools.partial(
    pl.kernel,
    out_type=jax.ShapeDtypeStruct((2, _NP, 128), jnp.float32),
    mesh=_sc_mesh,
    scratch_types=[
        pltpu.VMEM((_NCH, _CHUNK), jnp.int32),
        pltpu.VMEM((_SBUF, _CHUNK, 128), jnp.float32),
        pltpu.VMEM_SHARED((_NP, 128), jnp.float32),
        pltpu.SemaphoreType.DMA,
    ],
)(_scatter_body)


def kernel(node_attr, edge_index, edge_attr, edge_sh, fc_w1, fc_b1, fc_w2, fc_b2,
           bn_gamma, bn_beta):
    padn = _EP - _E
    dst = jnp.pad(edge_index[1], (0, padn))
    dst4 = (dst // 4).reshape(_NW, _NCH, _CHUNK)
    off = (dst % 4).astype(jnp.int32).reshape(_EP, 1)
    src = jnp.pad(edge_index[0], (0, padn)).reshape(_NW, _NCH, _CHUNK)
    ea_p = jnp.pad(edge_attr, ((0, padn), (0, 0)))
    sh_p = jnp.pad(edge_sh * (1.0 / jnp.sqrt(jnp.float32(_IN))), ((0, padn), (0, 0)))
    rep = jnp.repeat(jnp.eye(_IN, dtype=jnp.float32), _OUT, axis=1)
    zinit = jnp.zeros((_NP, 128), jnp.float32)
    node4 = node_attr.reshape(_N // 4, 128)

    xq = _gather_call(node4, dst4)

    tpc = pl.pallas_call(
        _tc_body,
        grid=(_EP // _TCB,),
        in_specs=[
            pl.BlockSpec((_TCB, _NEF), lambda i: (i, 0)),
            pl.BlockSpec((_TCB, 1), lambda i: (i, 0)),
            pl.BlockSpec((_TCB, 1), lambda i: (i, 0)),
            pl.BlockSpec((_TCB, 128), lambda i: (i, 0)),
            pl.BlockSpec((_NEF, _HID), lambda i: (0, 0)),
            pl.BlockSpec((1, _HID), lambda i: (0, 0)),
            pl.BlockSpec((_HID, _IN * _OUT), lambda i: (0, 0)),
            pl.BlockSpec((1, _IN * _OUT), lambda i: (0, 0)),
            pl.BlockSpec((_IN, _IN * _OUT), lambda i: (0, 0)),
        ],
        out_specs=pl.BlockSpec((_TCB, 128), lambda i: (i, 0)),
        out_shape=jax.ShapeDtypeStruct((_EP, 128), jnp.float32),
    )(ea_p, sh_p, off, xq, fc_w1, fc_b1.reshape(1, _HID), fc_w2,
      fc_b2.reshape(1, _IN * _OUT), rep)

    parts = _scatter_call(tpc, src, zinit)

    out = pl.pallas_call(
        _fin_body,
        out_shape=jax.ShapeDtypeStruct((_N, _OUT), jnp.float32),
    )(parts, node_attr, bn_gamma.reshape(1, _OUT), bn_beta.reshape(1, _OUT))
    return out


# speedup vs baseline: 1.2898x; 1.2898x over previous
import functools

import jax
import jax.numpy as jnp
from jax import lax
from jax.experimental import pallas as pl
from jax.experimental.pallas import tpu as pltpu
from jax.experimental.pallas import tpu_sc as plsc

_N = 10000
_E = 160000
_IN = 32
_OUT = 32
_NEF = 16
_HID = 64
_EPS = 1e-5

_NW = 32
_CHUNK = 128
_NCH = 40
_EPW = _CHUNK * _NCH
_EP = _EPW * _NW
_NBUF = 7
_SBUF = 2
_NP = 10240
_ROWS_PER_SUB = _NP // 16
_TCB = 640
_INV_SQRT_IN = float(1.0 / (32.0 ** 0.5))


def _gather_body(node4_hbm, dst4_hbm, x_hbm, idx_v, rows_v, sem_in, sem_out):
    c = lax.axis_index("c")
    s = lax.axis_index("s")
    wid = s * 2 + c
    pltpu.sync_copy(dst4_hbm.at[wid], idx_v)
    base = wid * _EPW
    cin = [None] * _NCH
    cout = [None] * _NCH
    for j in range(_NBUF):
        cin[j] = pltpu.async_copy(node4_hbm.at[idx_v.at[j]], rows_v.at[j % _NBUF],
                                  sem_in)
    for j in range(_NCH):
        cin[j].wait()
        cout[j] = pltpu.async_copy(
            rows_v.at[j % _NBUF], x_hbm.at[pl.ds(base + j * _CHUNK, _CHUNK)], sem_out)
        nj = j + _NBUF
        if nj < _NCH:
            cout[j].wait()
            cin[nj] = pltpu.async_copy(node4_hbm.at[idx_v.at[nj]],
                                       rows_v.at[nj % _NBUF], sem_in)
    for j in range(_NCH - _NBUF, _NCH):
        cout[j].wait()


def _scatter_body(tpc_hbm, src_hbm, zinit_hbm, out_hbm, idx_v, rows_v, acc_sh, sem):
    c = lax.axis_index("c")
    s = lax.axis_index("s")
    wid = s * 2 + c
    pltpu.sync_copy(zinit_hbm.at[pl.ds(s * _ROWS_PER_SUB, _ROWS_PER_SUB)],
                    acc_sh.at[pl.ds(s * _ROWS_PER_SUB, _ROWS_PER_SUB)])
    pltpu.sync_copy(src_hbm.at[wid], idx_v)
    plsc.subcore_barrier()
    base = wid * _EPW
    for g in range(_NCH // _SBUF):
        cps = []
        for b in range(_SBUF):
            j = g * _SBUF + b
            cps.append(pltpu.async_copy(
                tpc_hbm.at[pl.ds(base + j * _CHUNK, _CHUNK)], rows_v.at[b], sem))
        for b in range(_SBUF):
            j = g * _SBUF + b
            cps[b].wait()
            pltpu.sync_copy(rows_v.at[b], acc_sh.at[idx_v.at[j]], add=True)
    plsc.subcore_barrier()
    pltpu.sync_copy(acc_sh.at[pl.ds(s * _ROWS_PER_SUB, _ROWS_PER_SUB)],
                    out_hbm.at[c, pl.ds(s * _ROWS_PER_SUB, _ROWS_PER_SUB)])


def _tc_body(ea_ref, sh_ref, xq_ref, w1_ref, b1_ref, w2_ref, b2_ref,
             rep_ref, out_ref):
    x = xq_ref[:, :_IN]
    h = jnp.maximum(
        jnp.dot(ea_ref[...].astype(jnp.bfloat16), w1_ref[...],
                preferred_element_type=jnp.float32) + b1_ref[...], 0.0)
    wb = jnp.dot(h.astype(jnp.bfloat16), w2_ref[...],
                 preferred_element_type=jnp.float32) + b2_ref[...]
    xf = jnp.dot(x.astype(jnp.bfloat16), rep_ref[...],
                 preferred_element_type=jnp.float32)
    p = xf * wb
    p = p[:, :512] + p[:, 512:]
    p = p[:, :256] + p[:, 256:]
    p = p[:, :128] + p[:, 128:]
    p = p[:, :64] + p[:, 64:]
    p = p[:, :32] + p[:, 32:]
    i = pl.program_id(0)
    vmask = ((lax.broadcasted_iota(jnp.int32, (_TCB, 1), 0) + i * _TCB) < _E
             ).astype(jnp.float32)
    tp = p * (sh_ref[...] * (_INV_SQRT_IN * vmask))
    valid = jnp.broadcast_to(vmask, (_TCB, 16))
    out_ref[...] = jnp.concatenate([tp, valid, jnp.zeros((_TCB, 80), jnp.float32)], axis=1)


def _fin_body(parts_ref, node_ref, gamma_ref, beta_ref, out_ref):
    sums = parts_ref[0, :_N, :_OUT] + parts_ref[1, :_N, :_OUT]
    cnt = parts_ref[0, :_N, _OUT:_OUT + 1] + parts_ref[1, :_N, _OUT:_OUT + 1]
    o = sums / jnp.maximum(cnt, 1.0) + node_ref[...]
    mu = jnp.mean(o, axis=0, keepdims=True)
    var = jnp.mean((o - mu) ** 2, axis=0, keepdims=True)
    out_ref[...] = (o - mu) * lax.rsqrt(var + _EPS) * gamma_ref[...] + beta_ref[...]


_sc_mesh = plsc.VectorSubcoreMesh(core_axis_name="c", subcore_axis_name="s")

_gather_call = functools.partial(
    pl.kernel,
    out_type=jax.ShapeDtypeStruct((_EP, 128), jnp.float32),
    mesh=_sc_mesh,
    scratch_types=[
        pltpu.VMEM((_NCH, _CHUNK), jnp.int32),
        pltpu.VMEM((_NBUF, _CHUNK, 128), jnp.float32),
        pltpu.SemaphoreType.DMA,
        pltpu.SemaphoreType.DMA,
    ],
)(_gather_body)

_scatter_call = functools.partial(
    pl.kernel,
    out_type=jax.ShapeDtypeStruct((2, _NP, 128), jnp.float32),
    mesh=_sc_mesh,
    scratch_types=[
        pltpu.VMEM((_NCH, _CHUNK), jnp.int32),
        pltpu.VMEM((_SBUF, _CHUNK, 128), jnp.float32),
        pltpu.VMEM_SHARED((_NP, 128), jnp.float32),
        pltpu.SemaphoreType.DMA,
    ],
)(_scatter_body)


def kernel(node_attr, edge_index, edge_attr, edge_sh, fc_w1, fc_b1, fc_w2, fc_b2,
           bn_gamma, bn_beta):
    padn = _EP - _E
    dst4 = jnp.pad(edge_index[1], (0, padn)).reshape(_NW, _NCH, _CHUNK)
    src = jnp.pad(edge_index[0], (0, padn)).reshape(_NW, _NCH, _CHUNK)
    rep = jnp.repeat(jnp.eye(_IN, dtype=jnp.bfloat16), _OUT, axis=1)
    zinit = jnp.zeros((_NP, 128), jnp.float32)
    node4 = jnp.pad(node_attr, ((0, 0), (0, 128 - _IN)))

    xq = _gather_call(node4, dst4)

    nreal = _E // _TCB - 1
    tpc = pl.pallas_call(
        _tc_body,
        grid=(_EP // _TCB,),
        in_specs=[
            pl.BlockSpec((_TCB, _NEF), lambda i: (jnp.minimum(i, nreal), 0)),
            pl.BlockSpec((_TCB, 1), lambda i: (jnp.minimum(i, nreal), 0)),
            pl.BlockSpec((_TCB, 128), lambda i: (i, 0)),
            pl.BlockSpec((_NEF, _HID), lambda i: (0, 0)),
            pl.BlockSpec((1, _HID), lambda i: (0, 0)),
            pl.BlockSpec((_HID, _IN * _OUT), lambda i: (0, 0)),
            pl.BlockSpec((1, _IN * _OUT), lambda i: (0, 0)),
            pl.BlockSpec((_IN, _IN * _OUT), lambda i: (0, 0)),
        ],
        out_specs=pl.BlockSpec((_TCB, 128), lambda i: (i, 0)),
        out_shape=jax.ShapeDtypeStruct((_EP, 128), jnp.float32),
    )(edge_attr, edge_sh, xq, fc_w1.astype(jnp.bfloat16), fc_b1.reshape(1, _HID),
      fc_w2.astype(jnp.bfloat16), fc_b2.reshape(1, _IN * _OUT), rep)

    parts = _scatter_call(tpc, src, zinit)

    out = pl.pallas_call(
        _fin_body,
        out_shape=jax.ShapeDtypeStruct((_N, _OUT), jnp.float32),
    )(parts, node_attr, bn_gamma.reshape(1, _OUT), bn_beta.reshape(1, _OUT))
    return out


# speedup vs baseline: 1.3118x; 1.0170x over previous
import functools

import jax
import jax.numpy as jnp
from jax import lax
from jax.experimental import pallas as pl
from jax.experimental.pallas import tpu as pltpu
from jax.experimental.pallas import tpu_sc as plsc

_N = 10000
_E = 160000
_IN = 32
_OUT = 32
_NEF = 16
_HID = 64
_EPS = 1e-5

_NW = 32
_CHUNK = 128
_NCH = 40
_EPW = _CHUNK * _NCH
_EP = _EPW * _NW
_NBUF = 7
_SBUF = 2
_NP = 10240
_ROWS_PER_SUB = _NP // 16
_TCB = 640
_INV_SQRT_IN = float(1.0 / (32.0 ** 0.5))


def _gather_body(node4_hbm, dst4_hbm, x_hbm, idx_v, rows_v, sem_in, sem_out):
    c = lax.axis_index("c")
    s = lax.axis_index("s")
    wid = s * 2 + c
    pltpu.sync_copy(dst4_hbm.at[wid], idx_v)
    base = wid * _EPW
    cin = [None] * _NCH
    cout = [None] * _NCH
    for j in range(_NBUF):
        cin[j] = pltpu.async_copy(node4_hbm.at[idx_v.at[j]], rows_v.at[j % _NBUF],
                                  sem_in)
    for j in range(_NCH):
        cin[j].wait()
        cout[j] = pltpu.async_copy(
            rows_v.at[j % _NBUF], x_hbm.at[pl.ds(base + j * _CHUNK, _CHUNK)], sem_out)
        nj = j + _NBUF
        if nj < _NCH:
            cout[j].wait()
            cin[nj] = pltpu.async_copy(node4_hbm.at[idx_v.at[nj]],
                                       rows_v.at[nj % _NBUF], sem_in)
    for j in range(_NCH - _NBUF, _NCH):
        cout[j].wait()


def _scatter_body(tpc_hbm, src_hbm, zinit_hbm, out_hbm, idx_v, rows_v, acc_sh, sem):
    c = lax.axis_index("c")
    s = lax.axis_index("s")
    wid = s * 2 + c
    pltpu.sync_copy(zinit_hbm.at[pl.ds(s * _ROWS_PER_SUB, _ROWS_PER_SUB)],
                    acc_sh.at[pl.ds(s * _ROWS_PER_SUB, _ROWS_PER_SUB)])
    pltpu.sync_copy(src_hbm.at[wid], idx_v)
    plsc.subcore_barrier()
    base = wid * _EPW
    for g in range(_NCH // _SBUF):
        cps = []
        for b in range(_SBUF):
            j = g * _SBUF + b
            cps.append(pltpu.async_copy(
                tpc_hbm.at[pl.ds(base + j * _CHUNK, _CHUNK)], rows_v.at[b], sem))
        for b in range(_SBUF):
            j = g * _SBUF + b
            cps[b].wait()
            pltpu.sync_copy(rows_v.at[b], acc_sh.at[idx_v.at[j]], add=True)
    plsc.subcore_barrier()
    pltpu.sync_copy(acc_sh.at[pl.ds(s * _ROWS_PER_SUB, _ROWS_PER_SUB)],
                    out_hbm.at[c, pl.ds(s * _ROWS_PER_SUB, _ROWS_PER_SUB)])


def _tc_body(ea_ref, sh_ref, xq_ref, w1_ref, b1_ref, w2_ref, b2_ref,
             rep_ref, out_ref):
    x = xq_ref[:, :_IN]
    h = jnp.maximum(
        lax.dot_general(ea_ref[...].astype(jnp.bfloat16), w1_ref[...],
                        (((0,), (0,)), ((), ())),
                        preferred_element_type=jnp.float32) + b1_ref[...], 0.0)
    wb = jnp.dot(h.astype(jnp.bfloat16), w2_ref[...],
                 preferred_element_type=jnp.float32) + b2_ref[...]
    xf = jnp.dot(x.astype(jnp.bfloat16), rep_ref[...],
                 preferred_element_type=jnp.float32)
    p = xf * wb
    p = p[:, :512] + p[:, 512:]
    p = p[:, :256] + p[:, 256:]
    p = p[:, :128] + p[:, 128:]
    p = p[:, :64] + p[:, 64:]
    p = p[:, :32] + p[:, 32:]
    i = pl.program_id(0)
    vmask = ((lax.broadcasted_iota(jnp.int32, (_TCB, 1), 0) + i * _TCB) < _E
             ).astype(jnp.float32)
    shcol = lax.dot_general(sh_ref[...], jnp.ones((1, 1), jnp.float32),
                            (((0,), (0,)), ((), ())),
                            preferred_element_type=jnp.float32)
    tp = p * (shcol * (_INV_SQRT_IN * vmask))
    valid = jnp.broadcast_to(vmask, (_TCB, 16))
    out_ref[...] = jnp.concatenate([tp, valid, jnp.zeros((_TCB, 80), jnp.float32)], axis=1)


def _fin_body(parts_ref, node_ref, gamma_ref, beta_ref, out_ref):
    sums = parts_ref[0, :_N, :_OUT] + parts_ref[1, :_N, :_OUT]
    cnt = parts_ref[0, :_N, _OUT:_OUT + 1] + parts_ref[1, :_N, _OUT:_OUT + 1]
    o = sums / jnp.maximum(cnt, 1.0) + node_ref[...]
    mu = jnp.mean(o, axis=0, keepdims=True)
    var = jnp.mean((o - mu) ** 2, axis=0, keepdims=True)
    out_ref[...] = (o - mu) * lax.rsqrt(var + _EPS) * gamma_ref[...] + beta_ref[...]


_sc_mesh = plsc.VectorSubcoreMesh(core_axis_name="c", subcore_axis_name="s")

_gather_call = functools.partial(
    pl.kernel,
    out_type=jax.ShapeDtypeStruct((_EP, 128), jnp.float32),
    mesh=_sc_mesh,
    scratch_types=[
        pltpu.VMEM((_NCH, _CHUNK), jnp.int32),
        pltpu.VMEM((_NBUF, _CHUNK, 128), jnp.float32),
        pltpu.SemaphoreType.DMA,
        pltpu.SemaphoreType.DMA,
    ],
)(_gather_body)

_scatter_call = functools.partial(
    pl.kernel,
    out_type=jax.ShapeDtypeStruct((2, _NP, 128), jnp.float32),
    mesh=_sc_mesh,
    scratch_types=[
        pltpu.VMEM((_NCH, _CHUNK), jnp.int32),
        pltpu.VMEM((_SBUF, _CHUNK, 128), jnp.float32),
        pltpu.VMEM_SHARED((_NP, 128), jnp.float32),
        pltpu.SemaphoreType.DMA,
    ],
)(_scatter_body)


def kernel(node_attr, edge_index, edge_attr, edge_sh, fc_w1, fc_b1, fc_w2, fc_b2,
           bn_gamma, bn_beta):
    padn = _EP - _E
    dst4 = jnp.pad(edge_index[1], (0, padn)).reshape(_NW, _NCH, _CHUNK)
    src = jnp.pad(edge_index[0], (0, padn)).reshape(_NW, _NCH, _CHUNK)
    rep = jnp.repeat(jnp.eye(_IN, dtype=jnp.bfloat16), _OUT, axis=1)
    zinit = jnp.zeros((_NP, 128), jnp.float32)
    node4 = jnp.pad(node_attr, ((0, 0), (0, 128 - _IN)))

    xq = _gather_call(node4, dst4)

    nreal = _E // _TCB - 1
    tpc = pl.pallas_call(
        _tc_body,
        grid=(_EP // _TCB,),
        in_specs=[
            pl.BlockSpec((_NEF, _TCB), lambda i: (0, jnp.minimum(i, nreal))),
            pl.BlockSpec((1, _TCB), lambda i: (0, jnp.minimum(i, nreal))),
            pl.BlockSpec((_TCB, 128), lambda i: (i, 0)),
            pl.BlockSpec((_NEF, _HID), lambda i: (0, 0)),
            pl.BlockSpec((1, _HID), lambda i: (0, 0)),
            pl.BlockSpec((_HID, _IN * _OUT), lambda i: (0, 0)),
            pl.BlockSpec((1, _IN * _OUT), lambda i: (0, 0)),
            pl.BlockSpec((_IN, _IN * _OUT), lambda i: (0, 0)),
        ],
        out_specs=pl.BlockSpec((_TCB, 128), lambda i: (i, 0)),
        out_shape=jax.ShapeDtypeStruct((_EP, 128), jnp.float32),
    )(edge_attr.T, edge_sh.T, xq, fc_w1.astype(jnp.bfloat16),
      fc_b1.reshape(1, _HID), fc_w2.astype(jnp.bfloat16),
      fc_b2.reshape(1, _IN * _OUT), rep)

    parts = _scatter_call(tpc, src, zinit)

    out = pl.pallas_call(
        _fin_body,
        out_shape=jax.ShapeDtypeStruct((_N, _OUT), jnp.float32),
    )(parts, node_attr, bn_gamma.reshape(1, _OUT), bn_beta.reshape(1, _OUT))
    return out


# speedup vs baseline: 1.4235x; 1.0852x over previous
import functools

import jax
import jax.numpy as jnp
from jax import lax
from jax.experimental import pallas as pl
from jax.experimental.pallas import tpu as pltpu
from jax.experimental.pallas import tpu_sc as plsc

_N = 10000
_E = 160000
_IN = 32
_OUT = 32
_NEF = 16
_HID = 64
_EPS = 1e-5

_NSL = 2
_NW = 32
_CHUNK = 128
_NCH = 40 // _NSL
_EPW = _CHUNK * _NCH
_ESL = _EPW * _NW
_EP = _ESL * _NSL
_NBUF = 7
_SBUF = 2
_NP = 10240
_ROWS_PER_SUB = _NP // 16
_TCB = 640
_INV_SQRT_IN = float(1.0 / (32.0 ** 0.5))


def _gather_body(node_hbm, dst_hbm, x_hbm, idx_v, rows_v, sem_in, sem_out):
    c = lax.axis_index("c")
    s = lax.axis_index("s")
    wid = s * 2 + c
    pltpu.sync_copy(dst_hbm.at[wid], idx_v)
    base = wid * _EPW
    cin = [None] * _NCH
    cout = [None] * _NCH
    for j in range(_NBUF):
        cin[j] = pltpu.async_copy(node_hbm.at[idx_v.at[j]], rows_v.at[j % _NBUF],
                                  sem_in)
    for j in range(_NCH):
        cin[j].wait()
        cout[j] = pltpu.async_copy(
            rows_v.at[j % _NBUF], x_hbm.at[pl.ds(base + j * _CHUNK, _CHUNK)], sem_out)
        nj = j + _NBUF
        if nj < _NCH:
            cout[j].wait()
            cin[nj] = pltpu.async_copy(node_hbm.at[idx_v.at[nj]],
                                       rows_v.at[nj % _NBUF], sem_in)
    for j in range(_NCH - _NBUF, _NCH):
        cout[j].wait()


def _scatter_body(tpc_hbm, src_hbm, zinit_hbm, out_hbm, idx_v, rows_v, acc_sh, sem):
    c = lax.axis_index("c")
    s = lax.axis_index("s")
    wid = s * 2 + c
    pltpu.sync_copy(zinit_hbm.at[pl.ds(s * _ROWS_PER_SUB, _ROWS_PER_SUB)],
                    acc_sh.at[pl.ds(s * _ROWS_PER_SUB, _ROWS_PER_SUB)])
    pltpu.sync_copy(src_hbm.at[wid], idx_v)
    plsc.subcore_barrier()
    base = wid * _EPW
    for g in range(_NCH // _SBUF):
        cps = []
        for b in range(_SBUF):
            j = g * _SBUF + b
            cps.append(pltpu.async_copy(
                tpc_hbm.at[pl.ds(base + j * _CHUNK, _CHUNK)], rows_v.at[b], sem))
        for b in range(_SBUF):
            j = g * _SBUF + b
            cps[b].wait()
            pltpu.sync_copy(rows_v.at[b], acc_sh.at[idx_v.at[j]], add=True)
    plsc.subcore_barrier()
    pltpu.sync_copy(acc_sh.at[pl.ds(s * _ROWS_PER_SUB, _ROWS_PER_SUB)],
                    out_hbm.at[c, pl.ds(s * _ROWS_PER_SUB, _ROWS_PER_SUB)])


def _tc_body(ea_ref, sh_ref, xq_ref, w1_ref, b1_ref, w2_ref, rep_ref, out_ref,
             *, ebase):
    x = xq_ref[:, :_IN]
    h = jnp.maximum(
        lax.dot_general(ea_ref[...].astype(jnp.bfloat16), w1_ref[...],
                        (((0,), (0,)), ((), ())),
                        preferred_element_type=jnp.float32) + b1_ref[...], 0.0)
    h2 = jnp.concatenate(
        [h.astype(jnp.bfloat16), jnp.ones((_TCB, 1), jnp.bfloat16)], axis=1)
    wb = jnp.dot(h2, w2_ref[...], preferred_element_type=jnp.float32)
    xf = jnp.dot(x.astype(jnp.bfloat16), rep_ref[...],
                 preferred_element_type=jnp.float32)
    p = xf * wb
    p = p[:, :512] + p[:, 512:]
    p = p[:, :256] + p[:, 256:]
    p = p[:, :128] + p[:, 128:]
    p = p[:, :64] + p[:, 64:]
    p = p[:, :32] + p[:, 32:]
    i = pl.program_id(0)
    vmask = ((lax.broadcasted_iota(jnp.int32, (_TCB, 1), 0) + (i * _TCB + ebase))
             < _E).astype(jnp.float32)
    shcol = lax.dot_general(sh_ref[...], jnp.ones((1, 1), jnp.float32),
                            (((0,), (0,)), ((), ())),
                            preferred_element_type=jnp.float32)
    tp = p * (shcol * (_INV_SQRT_IN * vmask))
    valid = jnp.broadcast_to(vmask, (_TCB, 16))
    out_ref[...] = jnp.concatenate(
        [tp, valid, jnp.zeros((_TCB, 80), jnp.float32)], axis=1)


def _fin_body(p0_ref, p1_ref, node_ref, gamma_ref, beta_ref, out_ref):
    sums = (p0_ref[0, :_N, :_OUT] + p0_ref[1, :_N, :_OUT]
            + p1_ref[0, :_N, :_OUT] + p1_ref[1, :_N, :_OUT])
    cnt = (p0_ref[0, :_N, _OUT:_OUT + 1] + p0_ref[1, :_N, _OUT:_OUT + 1]
           + p1_ref[0, :_N, _OUT:_OUT + 1] + p1_ref[1, :_N, _OUT:_OUT + 1])
    o = sums / jnp.maximum(cnt, 1.0) + node_ref[...]
    mu = jnp.mean(o, axis=0, keepdims=True)
    var = jnp.mean((o - mu) ** 2, axis=0, keepdims=True)
    out_ref[...] = (o - mu) * lax.rsqrt(var + _EPS) * gamma_ref[...] + beta_ref[...]


_sc_mesh = plsc.VectorSubcoreMesh(core_axis_name="c", subcore_axis_name="s")

_gather_call = functools.partial(
    pl.kernel,
    out_type=jax.ShapeDtypeStruct((_ESL, 128), jnp.float32),
    mesh=_sc_mesh,
    scratch_types=[
        pltpu.VMEM((_NCH, _CHUNK), jnp.int32),
        pltpu.VMEM((_NBUF, _CHUNK, 128), jnp.float32),
        pltpu.SemaphoreType.DMA,
        pltpu.SemaphoreType.DMA,
    ],
)(_gather_body)

_scatter_call = functools.partial(
    pl.kernel,
    out_type=jax.ShapeDtypeStruct((2, _NP, 128), jnp.float32),
    mesh=_sc_mesh,
    scratch_types=[
        pltpu.VMEM((_NCH, _CHUNK), jnp.int32),
        pltpu.VMEM((_SBUF, _CHUNK, 128), jnp.float32),
        pltpu.VMEM_SHARED((_NP, 128), jnp.float32),
        pltpu.SemaphoreType.DMA,
    ],
)(_scatter_body)


def _tc_slice_call(eaT, shT, xq, w1, b1, w2b, rep, ebase):
    nblk = _ESL // _TCB
    first = ebase // _TCB
    nreal = _E // _TCB - 1
    body = functools.partial(_tc_body, ebase=ebase)

    def emap(i):
        return (0, jnp.minimum(i + first, nreal))

    return pl.pallas_call(
        body,
        grid=(nblk,),
        in_specs=[
            pl.BlockSpec((_NEF, _TCB), emap),
            pl.BlockSpec((1, _TCB), emap),
            pl.BlockSpec((_TCB, 128), lambda i: (i, 0)),
            pl.BlockSpec((_NEF, _HID), lambda i: (0, 0)),
            pl.BlockSpec((1, _HID), lambda i: (0, 0)),
            pl.BlockSpec((_HID + 1, _IN * _OUT), lambda i: (0, 0)),
            pl.BlockSpec((_IN, _IN * _OUT), lambda i: (0, 0)),
        ],
        out_specs=pl.BlockSpec((_TCB, 128), lambda i: (i, 0)),
        out_shape=jax.ShapeDtypeStruct((_ESL, 128), jnp.float32),
    )(eaT, shT, xq, w1, b1, w2b, rep)


def kernel(node_attr, edge_index, edge_attr, edge_sh, fc_w1, fc_b1, fc_w2, fc_b2,
           bn_gamma, bn_beta):
    padn = _EP - _E
    dst4 = jnp.pad(edge_index[1], (0, padn)).reshape(_NSL, _NW, _NCH, _CHUNK)
    src = jnp.pad(edge_index[0], (0, padn)).reshape(_NSL, _NW, _NCH, _CHUNK)
    rep = jnp.repeat(jnp.eye(_IN, dtype=jnp.bfloat16), _OUT, axis=1)
    zinit = jnp.zeros((_NP, 128), jnp.float32)
    node4 = jnp.pad(node_attr, ((0, 0), (0, 128 - _IN)))
    w2b = jnp.concatenate([fc_w2, fc_b2[None, :]], axis=0).astype(jnp.bfloat16)
    w1c = fc_w1.astype(jnp.bfloat16)
    b1r = fc_b1.reshape(1, _HID)
    eaT = edge_attr.T
    shT = edge_sh.T

    parts = []
    for k in range(_NSL):
        xq = _gather_call(node4, dst4[k])
        tpc = _tc_slice_call(eaT, shT, xq, w1c, b1r, w2b, rep, k * _ESL)
        parts.append(_scatter_call(tpc, src[k], zinit))

    out = pl.pallas_call(
        _fin_body,
        out_shape=jax.ShapeDtypeStruct((_N, _OUT), jnp.float32),
    )(parts[0], parts[1], node_attr, bn_gamma.reshape(1, _OUT),
      bn_beta.reshape(1, _OUT))
    return out


# speedup vs baseline: 1.7810x; 1.2511x over previous
import functools

import jax
import jax.numpy as jnp
from jax import lax
from jax.experimental import pallas as pl
from jax.experimental.pallas import tpu as pltpu
from jax.experimental.pallas import tpu_sc as plsc

_N = 10000
_E = 160000
_IN = 32
_OUT = 32
_NEF = 16
_HID = 64
_EPS = 1e-5

_NSL = 2
_NW = 32
_CHUNK = 128
_NCH = 40 // _NSL
_EPW = _CHUNK * _NCH
_ESL = _EPW * _NW
_EP = _ESL * _NSL
_NBUF = 7
_SBUF = 2
_NP = 10240
_ROWS_PER_SUB = _NP // 16
_TCB = 640
_INV_SQRT_IN = float(1.0 / (32.0 ** 0.5))


def _gather_body(node_hbm, dst_hbm, x_hbm, idx_v, rows_v, sem_in, sem_out):
    c = lax.axis_index("c")
    s = lax.axis_index("s")
    wid = s * 2 + c
    pltpu.sync_copy(dst_hbm.at[wid], idx_v)
    base = wid * _EPW
    cin = [None] * _NCH
    cout = [None] * _NCH
    for j in range(_NBUF):
        cin[j] = pltpu.async_copy(node_hbm.at[idx_v.at[j]], rows_v.at[j % _NBUF],
                                  sem_in)
    for j in range(_NCH):
        cin[j].wait()
        cout[j] = pltpu.async_copy(
            rows_v.at[j % _NBUF], x_hbm.at[pl.ds(base + j * _CHUNK, _CHUNK)], sem_out)
        nj = j + _NBUF
        if nj < _NCH:
            cout[j].wait()
            cin[nj] = pltpu.async_copy(node_hbm.at[idx_v.at[nj]],
                                       rows_v.at[nj % _NBUF], sem_in)
    for j in range(_NCH - _NBUF, _NCH):
        cout[j].wait()


def _scatter_body(tpc_hbm, src_hbm, zinit_hbm, out_hbm, idx_v, rows_v, acc_sh, sem):
    c = lax.axis_index("c")
    s = lax.axis_index("s")
    wid = s * 2 + c
    pltpu.sync_copy(zinit_hbm.at[pl.ds(s * _ROWS_PER_SUB, _ROWS_PER_SUB)],
                    acc_sh.at[pl.ds(s * _ROWS_PER_SUB, _ROWS_PER_SUB)])
    pltpu.sync_copy(src_hbm.at[wid], idx_v)
    plsc.subcore_barrier()
    base = wid * _EPW
    for g in range(_NCH // _SBUF):
        cps = []
        for b in range(_SBUF):
            j = g * _SBUF + b
            cps.append(pltpu.async_copy(
                tpc_hbm.at[pl.ds(base + j * _CHUNK, _CHUNK)], rows_v.at[b], sem))
        for b in range(_SBUF):
            j = g * _SBUF + b
            cps[b].wait()
            pltpu.sync_copy(rows_v.at[b], acc_sh.at[idx_v.at[j]], add=True)
    plsc.subcore_barrier()
    pltpu.sync_copy(acc_sh.at[pl.ds(s * _ROWS_PER_SUB, _ROWS_PER_SUB)],
                    out_hbm.at[c, pl.ds(s * _ROWS_PER_SUB, _ROWS_PER_SUB)])


def _tc_body(ea_ref, sh_ref, xq_ref, w1_ref, b1_ref, w2_ref, rep_ref, out_ref,
             *, ebase):
    x = xq_ref[:, :_IN]
    h = jnp.maximum(
        lax.dot_general(ea_ref[...].astype(jnp.bfloat16), w1_ref[...],
                        (((0,), (0,)), ((), ())),
                        preferred_element_type=jnp.float32) + b1_ref[...], 0.0)
    h2 = jnp.concatenate(
        [h.astype(jnp.bfloat16), jnp.ones((_TCB, 1), jnp.bfloat16)], axis=1)
    wb = jnp.dot(h2, w2_ref[...], preferred_element_type=jnp.float32)
    xf = jnp.dot(x.astype(jnp.bfloat16), rep_ref[...],
                 preferred_element_type=jnp.float32)
    p = xf * wb
    p = p[:, :512] + p[:, 512:]
    p = p[:, :256] + p[:, 256:]
    p = p[:, :128] + p[:, 128:]
    p = p[:, :64] + p[:, 64:]
    p = p[:, :32] + p[:, 32:]
    i = pl.program_id(0)
    vmask = ((lax.broadcasted_iota(jnp.int32, (_TCB, 1), 0) + (i * _TCB + ebase))
             < _E).astype(jnp.float32)
    shcol = lax.dot_general(sh_ref[...], jnp.ones((1, 1), jnp.float32),
                            (((0,), (0,)), ((), ())),
                            preferred_element_type=jnp.float32)
    tp = p * (shcol * (_INV_SQRT_IN * vmask))
    valid = jnp.broadcast_to(vmask, (_TCB, 16))
    out_ref[...] = jnp.concatenate(
        [tp, valid, jnp.zeros((_TCB, 80), jnp.float32)], axis=1)


def _fin_body(p0_ref, p1_ref, node_ref, gamma_ref, beta_ref, out_ref):
    sums = (p0_ref[0, :_N, :_OUT] + p0_ref[1, :_N, :_OUT]
            + p1_ref[0, :_N, :_OUT] + p1_ref[1, :_N, :_OUT])
    cnt = (p0_ref[0, :_N, _OUT:_OUT + 1] + p0_ref[1, :_N, _OUT:_OUT + 1]
           + p1_ref[0, :_N, _OUT:_OUT + 1] + p1_ref[1, :_N, _OUT:_OUT + 1])
    o = sums / jnp.maximum(cnt, 1.0) + node_ref[...]
    mu = jnp.mean(o, axis=0, keepdims=True)
    var = jnp.mean((o - mu) ** 2, axis=0, keepdims=True)
    out_ref[...] = (o - mu) * lax.rsqrt(var + _EPS) * gamma_ref[...] + beta_ref[...]


_sc_mesh = plsc.VectorSubcoreMesh(core_axis_name="c", subcore_axis_name="s")

_gather_call = functools.partial(
    pl.kernel,
    out_type=jax.ShapeDtypeStruct((_ESL, 128), jnp.float32),
    mesh=_sc_mesh,
    scratch_types=[
        pltpu.VMEM((_NCH, _CHUNK), jnp.int32),
        pltpu.VMEM((_NBUF, _CHUNK, 128), jnp.float32),
        pltpu.SemaphoreType.DMA,
        pltpu.SemaphoreType.DMA,
    ],
)(_gather_body)

_scatter_call = functools.partial(
    pl.kernel,
    out_type=jax.ShapeDtypeStruct((2, _NP, 128), jnp.float32),
    mesh=_sc_mesh,
    scratch_types=[
        pltpu.VMEM((_NCH, _CHUNK), jnp.int32),
        pltpu.VMEM((_SBUF, _CHUNK, 128), jnp.float32),
        pltpu.VMEM_SHARED((_NP, 128), jnp.float32),
        pltpu.SemaphoreType.DMA,
    ],
)(_scatter_body)


def _tc_slice_call(eaT, shT, xq, w1, b1, w2b, rep, ebase):
    nblk = _ESL // _TCB
    first = ebase // _TCB
    nreal = _E // _TCB - 1
    body = functools.partial(_tc_body, ebase=ebase)

    def emap(i):
        return (0, jnp.minimum(i + first, nreal))

    return pl.pallas_call(
        body,
        grid=(nblk,),
        in_specs=[
            pl.BlockSpec((_NEF, _TCB), emap),
            pl.BlockSpec((1, _TCB), emap),
            pl.BlockSpec((_TCB, 128), lambda i: (i, 0)),
            pl.BlockSpec((_NEF, _HID), lambda i: (0, 0)),
            pl.BlockSpec((1, _HID), lambda i: (0, 0)),
            pl.BlockSpec((_HID + 1, _IN * _OUT), lambda i: (0, 0)),
            pl.BlockSpec((_IN, _IN * _OUT), lambda i: (0, 0)),
        ],
        out_specs=pl.BlockSpec((_TCB, 128), lambda i: (i, 0)),
        out_shape=jax.ShapeDtypeStruct((_ESL, 128), jnp.float32),
    )(eaT, shT, xq, w1, b1, w2b, rep)


def kernel(node_attr, edge_index, edge_attr, edge_sh, fc_w1, fc_b1, fc_w2, fc_b2,
           bn_gamma, bn_beta):
    padn = _EP - _E
    dst4 = jnp.pad(edge_index[1], (0, padn)).reshape(_NSL, _NW, _NCH, _CHUNK)
    src = jnp.pad(edge_index[0], (0, padn)).reshape(_NSL, _NW, _NCH, _CHUNK)
    rep = jnp.repeat(jnp.eye(_IN, dtype=jnp.bfloat16), _OUT, axis=1)
    zinit = jnp.zeros((_NP, 128), jnp.float32)
    node4 = jnp.pad(node_attr, ((0, 0), (0, 128 - _IN)))
    w2b = jnp.concatenate([fc_w2, fc_b2[None, :]], axis=0).astype(jnp.bfloat16)
    w1c = fc_w1.astype(jnp.bfloat16)
    b1r = fc_b1.reshape(1, _HID)
    eaT = edge_attr.T
    shT = edge_sh.T

    parts = []
    dep = jnp.zeros((), jnp.int32)
    for k in range(_NSL):
        xq = _gather_call(node4, dst4[k] + dep)
        dep = (xq[0, 0] == jnp.inf).astype(jnp.int32)
        tpc = _tc_slice_call(eaT, shT, xq, w1c, b1r, w2b, rep, k * _ESL)
        parts.append(_scatter_call(tpc, src[k], zinit))

    out = pl.pallas_call(
        _fin_body,
        out_shape=jax.ShapeDtypeStruct((_N, _OUT), jnp.float32),
    )(parts[0], parts[1], node_attr, bn_gamma.reshape(1, _OUT),
      bn_beta.reshape(1, _OUT))
    return out
